# baseline (speedup 1.0000x reference)
import numpy as np

import jax
import jax.numpy as jnp
from jax.experimental import pallas as pl
from jax.experimental.pallas import tpu as pltpu

LANE = 128
KH = 5
S1 = 16
S2 = 8
BT = 8
T = 64


def _lane_stack(a, rows, t):
    return jnp.concatenate([a[i * rows:(i + 1) * rows, :] for i in range(t)],
                           axis=1)


def _m_stack(b, rows, d, t):
    blk = b[d * rows:(d + 1) * rows, :]
    return jnp.concatenate([blk[:, i * LANE:(i + 1) * LANE] for i in range(t)],
                           axis=0)


CHUNKS = 1
TC = T // CHUNKS


def _fused_body(x_ref, scat1_ref, cb1_ref, wcat1_ref, pp1_ref,
                scat2_ref, cb2_ref, wcat2_ref, pp2_ref,
                selcat_ref, fw1cat_ref, fb1_ref, fw2_ref, fb2_ref,
                fw3_ref, fb3_ref, o_ref):
    f32, bf16 = jnp.float32, jnp.bfloat16

    def mm(a, b):
        return jnp.dot(a, b, preferred_element_type=f32)

    n32 = BT * 32
    n16 = BT * S1
    n8 = BT * S2

    def chain(a):
        at = _lane_stack(a, n32, TC)
        b1 = mm(scat1_ref[...], at).astype(bf16)
        acat = jnp.concatenate(
            [a] + [_m_stack(b1, n32, d, TC) for d in range(KH - 1)], axis=1)
        acc1 = jnp.maximum(mm(acat, wcat1_ref[...]) + cb1_ref[...], 0.0)
        z1 = jnp.maximum(acc1[:, :LANE], acc1[:, LANE:]).astype(bf16)

        z1t = _lane_stack(z1, n32, TC)
        p1 = mm(pp1_ref[...], z1t).astype(bf16)
        a1t = jnp.maximum(p1[:n16, :], p1[n16:, :])

        b2 = mm(scat2_ref[...], a1t).astype(bf16)
        acat2 = jnp.concatenate(
            [_m_stack(a1t, n16, 0, TC)] +
            [_m_stack(b2, n16, d, TC) for d in range(KH - 1)], axis=1)
        acc2 = jnp.maximum(mm(acat2, wcat2_ref[...]) + cb2_ref[...], 0.0)
        z2 = jnp.maximum(acc2[:, :LANE], acc2[:, LANE:]).astype(bf16)

        z2t = _lane_stack(z2, n16, TC)
        p2 = mm(pp2_ref[...], z2t).astype(bf16)
        a2t = jnp.maximum(p2[:n8, :], p2[n8:, :])

        g = mm(selcat_ref[...], a2t)
        hcat = jnp.concatenate(
            [_m_stack(g, BT, r, TC) for r in range(KH)], axis=1)
        h = mm(hcat.astype(bf16), fw1cat_ref[...])
        h = jnp.maximum(h + fb1_ref[...], 0.0).astype(bf16)
        h = jnp.maximum(mm(h, fw2_ref[...]) + fb2_ref[...], 0.0).astype(bf16)
        return (mm(h, fw3_ref[...]) + fb3_ref[...]).astype(o_ref.dtype)

    rows_c, out_c = TC * n32, TC * BT
    for ci in range(CHUNKS):
        o_ref[ci * out_c:(ci + 1) * out_c, :] = chain(
            x_ref[ci * rows_c:(ci + 1) * rows_c, :])


def kernel(cw1, cb1, sh1, pe1, po1, cw2, cb2, sh2, pe2, po2,
           fw1, fb1, selh, fw2, fb2, fw3, fb3, x):
    bf16 = jnp.bfloat16
    n32, n16, n8 = BT * 32, BT * S1, BT * S2

    perm128 = np.arange(LANE, dtype=np.int32)
    for c in range(3):
        for w in range(32):
            perm128[c * 32 + w] = w * 3 + c

    scat1 = sh1.reshape(4 * n32, n32)
    wcat1 = cw1[:, perm128, :].reshape(KH * LANE, 2 * LANE)
    scat2 = sh2.reshape(4 * n16, n16)
    wcat2 = cw2.reshape(KH * LANE, 2 * LANE)
    pp1 = jnp.concatenate([pe1, po1], axis=0)
    pp2 = jnp.concatenate([pe2, po2], axis=0)
    selcat = selh.reshape(KH * BT, n8)
    fw1cat = fw1.reshape(KH * LANE, LANE)

    n = x.shape[0]
    step = BT * T
    n_pad = ((n + step - 1) // step) * step
    xs = jnp.transpose(x.astype(bf16), (0, 2, 1, 3)).reshape(n, 32, 32 * 3)
    xs = jnp.pad(xs, ((0, n_pad - n), (0, 0), (0, LANE - 32 * 3)))
    x2d = xs.reshape(n_pad * 32, LANE)

    rows = T * n32

    def resident(a):
        nd = a.ndim
        return pl.BlockSpec(tuple(a.shape), lambda i, nd=nd: (0,) * nd)

    ops = (scat1, cb1, wcat1, pp1, scat2, cb2, wcat2, pp2,
           selcat, fw1cat, fb1, fw2, fb2, fw3, fb3)
    out = pl.pallas_call(
        _fused_body,
        grid=(n_pad // step,),
        in_specs=[pl.BlockSpec((rows, LANE), lambda i: (i, 0))] +
                 [resident(a) for a in ops],
        out_specs=pl.BlockSpec((step, LANE), lambda i: (i, 0)),
        out_shape=jax.ShapeDtypeStruct((n_pad, LANE), jnp.float32),
        compiler_params=pltpu.CompilerParams(
            dimension_semantics=("arbitrary",)),
    )(x2d, *ops)
    return out[:n, :10]


# speedup vs baseline: 1.0035x; 1.0035x over previous
import numpy as np

import jax
import jax.numpy as jnp
from jax.experimental import pallas as pl
from jax.experimental.pallas import tpu as pltpu

LANE = 128
KH = 5
S1 = 16
S2 = 8
BT = 8
T = 64


def _lane_stack(a, rows, t):
    return jnp.concatenate([a[i * rows:(i + 1) * rows, :] for i in range(t)],
                           axis=1)


def _m_stack(b, rows, d, t):
    blk = b[d * rows:(d + 1) * rows, :]
    return jnp.concatenate([blk[:, i * LANE:(i + 1) * LANE] for i in range(t)],
                           axis=0)


CHUNKS = 1
TC = T // CHUNKS


def _fused_body(x_ref, scat1_ref, cb1_ref, wcat1_ref, pp1_ref,
                scat2_ref, cb2_ref, wcat2_ref, pp2_ref,
                selcat_ref, fw1cat_ref, fb1_ref, fw2_ref, fb2_ref,
                fw3_ref, fb3_ref, o_ref):
    f32, bf16 = jnp.float32, jnp.bfloat16

    def mm(a, b):
        return jnp.dot(a, b, preferred_element_type=f32)

    n32 = BT * 32
    n16 = BT * S1
    n8 = BT * S2

    def chain(a):
        at = _lane_stack(a, n32, TC)
        b1 = mm(scat1_ref[...], at).astype(bf16)
        acat = jnp.concatenate(
            [a] + [_m_stack(b1, n32, d, TC) for d in range(KH - 1)], axis=1)
        acc1 = jnp.maximum(mm(acat, wcat1_ref[...]) + cb1_ref[...], 0.0)
        z1 = jnp.maximum(acc1[:, :LANE], acc1[:, LANE:]).astype(bf16)

        z1t = _lane_stack(z1, n32, TC)
        p1 = mm(pp1_ref[...], z1t).astype(bf16)
        a1t = jnp.maximum(p1[:n16, :], p1[n16:, :])

        b2 = mm(scat2_ref[...], a1t).astype(bf16)
        acat2 = jnp.concatenate(
            [_m_stack(a1t, n16, 0, TC)] +
            [_m_stack(b2, n16, d, TC) for d in range(KH - 1)], axis=1)
        acc2 = jnp.maximum(mm(acat2, wcat2_ref[...]) + cb2_ref[...], 0.0)
        z2 = jnp.maximum(acc2[:, :LANE], acc2[:, LANE:]).astype(bf16)

        z2t = _lane_stack(z2, n16, TC)
        p2 = mm(pp2_ref[...], z2t).astype(bf16)
        a2t = jnp.maximum(p2[:n8, :], p2[n8:, :])

        g = mm(selcat_ref[...], a2t)
        hcat = jnp.concatenate(
            [_m_stack(g, BT, r, TC) for r in range(KH)], axis=1)
        h = mm(hcat.astype(bf16), fw1cat_ref[...])
        h = jnp.maximum(h + fb1_ref[...], 0.0).astype(bf16)
        h = jnp.maximum(mm(h, fw2_ref[...]) + fb2_ref[...], 0.0).astype(bf16)
        return (mm(h, fw3_ref[...]) + fb3_ref[...]).astype(o_ref.dtype)

    rows_c, out_c = TC * n32, TC * BT
    for ci in range(CHUNKS):
        o_ref[ci * out_c:(ci + 1) * out_c, :] = chain(
            x_ref[ci * rows_c:(ci + 1) * rows_c, :])


def kernel(cw1, cb1, sh1, pe1, po1, cw2, cb2, sh2, pe2, po2,
           fw1, fb1, selh, fw2, fb2, fw3, fb3, x):
    bf16 = jnp.bfloat16
    n32, n16, n8 = BT * 32, BT * S1, BT * S2

    perm128 = np.arange(LANE, dtype=np.int32)
    for c in range(3):
        for w in range(32):
            perm128[c * 32 + w] = w * 3 + c

    scat1 = sh1.reshape(4 * n32, n32)
    wcat1 = cw1[:, perm128, :].reshape(KH * LANE, 2 * LANE)
    scat2 = sh2.reshape(4 * n16, n16)
    wcat2 = cw2.reshape(KH * LANE, 2 * LANE)
    pp1 = jnp.concatenate([pe1, po1], axis=0)
    pp2 = jnp.concatenate([pe2, po2], axis=0)
    selcat = selh.reshape(KH * BT, n8)
    fw1cat = fw1.reshape(KH * LANE, LANE)

    n = x.shape[0]
    step = BT * T
    n_pad = ((n + step - 1) // step) * step
    xs = jnp.transpose(x.astype(bf16), (0, 2, 1, 3)).reshape(n, 32, 32 * 3)
    xs = jnp.pad(xs, ((0, n_pad - n), (0, 0), (0, LANE - 32 * 3)))
    x2d = xs.reshape(n_pad * 32, LANE)

    rows = T * n32

    def resident(a):
        nd = a.ndim
        return pl.BlockSpec(tuple(a.shape), lambda i, nd=nd: (0,) * nd)

    ops = (scat1, cb1, wcat1, pp1, scat2, cb2, wcat2, pp2,
           selcat, fw1cat, fb1, fw2, fb2, fw3, fb3)
    out = pl.pallas_call(
        _fused_body,
        grid=(n_pad // step,),
        in_specs=[pl.BlockSpec((rows, LANE), lambda i: (i, 0))] +
                 [resident(a) for a in ops],
        out_specs=pl.BlockSpec((step, LANE), lambda i: (i, 0)),
        out_shape=jax.ShapeDtypeStruct((n_pad, LANE), jnp.float32),
        compiler_params=pltpu.CompilerParams(
            dimension_semantics=("parallel",)),
    )(x2d, *ops)
    return out[:n, :10]
